# baseline (speedup 1.0000x reference)
import functools

import jax
import jax.numpy as jnp
from jax import lax
from jax.experimental import pallas as pl
from jax.experimental.pallas import tpu as pltpu
from jax.experimental.pallas import tpu_sc as plsc

B = 4096
L = 200
DIM = 64
N = B * L
NK = DIM // 16

NC = 2
NS = 16
NW = NC * NS
ROWS_W = B // NW

_mesh = plsc.VectorSubcoreMesh(core_axis_name="c", subcore_axis_name="s")


@functools.partial(
    pl.kernel,
    out_type=jax.ShapeDtypeStruct((B, L, DIM), jnp.float32),
    mesh=_mesh,
    compiler_params=pltpu.CompilerParams(use_tc_tiling_on_sc=False),
    scratch_types=[
        [pltpu.VMEM((L,), jnp.int32)] * 2,
        [pltpu.VMEM((L,), jnp.int32)] * 2,
        [pltpu.VMEM((L, DIM), jnp.float32)] * 2,
        [pltpu.VMEM((L, DIM), jnp.float32)] * 2,
        [pltpu.VMEM((L, DIM), jnp.float32)] * 2,
        pltpu.VMEM_SHARED((L, DIM), jnp.float32),
        pltpu.VMEM((DIM,), jnp.float32),
        pltpu.VMEM((DIM,), jnp.float32),
        [pltpu.SemaphoreType.DMA] * 2,
        [pltpu.SemaphoreType.DMA] * 2,
        [pltpu.SemaphoreType.DMA] * 2,
    ],
)
def _embed_ln_kernel(tok_hbm, pos_hbm, ttab_hbm, ptab_hbm, gamma_hbm, beta_hbm,
                     out_hbm,
                     idxt, idxp, trows, prows, orows, ptab_sh,
                     gamma_v, beta_v, sem_t, sem_p, sem_o):
    sid = lax.axis_index("s")
    wid = sid * NC + lax.axis_index("c")
    base_b = wid * ROWS_W

    pltpu.sync_copy(gamma_hbm, gamma_v)
    pltpu.sync_copy(beta_hbm, beta_v)

    @pl.when(sid == 0)
    def _():
        pltpu.sync_copy(ptab_hbm.at[pl.ds(0, L)], prows[0])
        pltpu.sync_copy(prows[0], ptab_sh)

    plsc.subcore_barrier()

    g = [gamma_v[pl.ds(k * 16, 16)] for k in range(NK)]
    bt = [beta_v[pl.ds(k * 16, 16)] for k in range(NK)]

    lane = lax.iota(jnp.int32, 16)
    perms = [lane ^ sh for sh in (1, 2, 4, 8)]

    def allsum(v):
        for p in perms:
            v = v + v.at[p].get(mode="promise_in_bounds")
        return v

    def fetch(gi, b):
        bi = base_b + gi
        pltpu.sync_copy(tok_hbm.at[bi], idxt[b])
        pltpu.sync_copy(pos_hbm.at[bi], idxp[b])
        pltpu.async_copy(ttab_hbm.at[idxt[b]], trows[b], sem_t[b])
        pltpu.async_copy(ptab_sh.at[idxp[b]], prows[b], sem_p[b])

    def compute_chunk(b):
        @plsc.parallel_loop(0, L, unroll=4)
        def tok_body(t):
            e = [trows[b][t, pl.ds(k * 16, 16)] + prows[b][t, pl.ds(k * 16, 16)]
                 for k in range(NK)]
            s = (e[0] + e[1]) + (e[2] + e[3])
            q = (e[0] * e[0] + e[1] * e[1]) + (e[2] * e[2] + e[3] * e[3])
            mean = allsum(s) * (1.0 / DIM)
            var = allsum(q) * (1.0 / DIM) - mean * mean
            xv = jnp.maximum(var, 0.0) + 1e-12
            iv = lax.bitcast_convert_type(xv, jnp.int32)
            iv = 0x5F3759DF - (iv >> 1)
            y = lax.bitcast_convert_type(iv, jnp.float32)
            hx = xv * 0.5
            for _ in range(3):
                y = y * (1.5 - hx * y * y)
            for k in range(NK):
                orows[b][t, pl.ds(k * 16, 16)] = \
                    (e[k] - mean) * y * g[k] + bt[k]

    fetch(0, 0)

    def body2(ch, carry):
        for b in (0, 1):
            gi = 2 * ch + b
            nb = 1 - b

            @pl.when(gi + 1 < ROWS_W)
            def _():
                fetch(gi + 1, nb)

            pltpu.make_async_copy(ttab_hbm.at[idxt[b]], trows[b],
                                  sem_t[b]).wait()
            pltpu.make_async_copy(ptab_sh.at[idxp[b]], prows[b],
                                  sem_p[b]).wait()

            @pl.when(gi >= 2)
            def _():
                pltpu.make_async_copy(orows[b], out_hbm.at[base_b + gi],
                                      sem_o[b]).wait()

            compute_chunk(b)
            pltpu.async_copy(orows[b], out_hbm.at[base_b + gi], sem_o[b])
        return carry

    lax.fori_loop(0, ROWS_W // 2, body2, 0)

    for b in (0, 1):
        pltpu.make_async_copy(orows[b], out_hbm.at[base_b], sem_o[b]).wait()


def kernel(input_token_id, input_position_id, token_table, pos_table,
           ln_gamma, ln_beta):
    tok = jnp.asarray(input_token_id, jnp.int32)
    pos = jnp.asarray(input_position_id, jnp.int32)
    return _embed_ln_kernel(tok, pos, token_table, pos_table,
                            ln_gamma, ln_beta)


# speedup vs baseline: 1.2779x; 1.2779x over previous
import functools

import jax
import jax.numpy as jnp
from jax import lax
from jax.experimental import pallas as pl
from jax.experimental.pallas import tpu as pltpu
from jax.experimental.pallas import tpu_sc as plsc

B = 4096
L = 200
DIM = 64
PAD = 128
N = B * L
NK = DIM // 16

NC = 2
NS = 16
NW = NC * NS
ROWS_W = B // NW

_mesh = plsc.VectorSubcoreMesh(core_axis_name="c", subcore_axis_name="s")


@functools.partial(
    pl.kernel,
    out_type=jax.ShapeDtypeStruct((B, L, PAD), jnp.float32),
    mesh=_mesh,
    compiler_params=pltpu.CompilerParams(use_tc_tiling_on_sc=False),
    scratch_types=[
        [pltpu.VMEM((L,), jnp.int32)] * 2,
        [pltpu.VMEM((L,), jnp.int32)] * 2,
        [pltpu.VMEM((L, PAD), jnp.float32)] * 2,
        pltpu.VMEM_SHARED((L, PAD), jnp.float32),
        pltpu.VMEM((DIM,), jnp.float32),
        pltpu.VMEM((DIM,), jnp.float32),
        [pltpu.SemaphoreType.DMA] * 2,
        [pltpu.SemaphoreType.DMA] * 2,
    ],
)
def _embed_ln_kernel(tok_hbm, pos_hbm, ttab_hbm, ptab_hbm, gamma_hbm, beta_hbm,
                     out_hbm,
                     idxt, idxp, erows, ptab_sh,
                     gamma_v, beta_v, sem_t, sem_o):
    sid = lax.axis_index("s")
    wid = sid * NC + lax.axis_index("c")
    base_b = wid * ROWS_W

    pltpu.sync_copy(gamma_hbm, gamma_v)
    pltpu.sync_copy(beta_hbm, beta_v)

    @pl.when(sid == 0)
    def _():
        pltpu.sync_copy(ptab_hbm, erows[0])
        pltpu.sync_copy(erows[0], ptab_sh)

    plsc.subcore_barrier()

    g = [gamma_v[pl.ds(k * 16, 16)] for k in range(NK)]
    bt = [beta_v[pl.ds(k * 16, 16)] for k in range(NK)]

    lane = lax.iota(jnp.int32, 16)
    perms = [lane ^ sh for sh in (1, 2, 4, 8)]

    def allsum(v):
        for p in perms:
            v = v + v.at[p].get(mode="promise_in_bounds")
        return v

    def fetch(gi, b):
        bi = base_b + gi
        pltpu.sync_copy(tok_hbm.at[pl.ds(bi * L, L)], idxt[b])
        pltpu.sync_copy(pos_hbm.at[pl.ds(bi * L, L)], idxp[b])

        @pl.when(gi >= 2)
        def _():
            pltpu.make_async_copy(erows[b], out_hbm.at[bi], sem_o[b]).wait()

        pltpu.sync_copy(ptab_sh.at[idxp[b]], erows[b])
        pltpu.async_copy(ttab_hbm.at[idxt[b]], erows[b], sem_t[b], add=True)

    def compute_chunk(b):
        @plsc.parallel_loop(0, L, unroll=4)
        def tok_body(t):
            e = [erows[b][t, pl.ds(k * 16, 16)] for k in range(NK)]
            s = (e[0] + e[1]) + (e[2] + e[3])
            q = (e[0] * e[0] + e[1] * e[1]) + (e[2] * e[2] + e[3] * e[3])
            mean = allsum(s) * (1.0 / DIM)
            var = allsum(q) * (1.0 / DIM) - mean * mean
            xv = jnp.maximum(var, 0.0) + 1e-12
            iv = lax.bitcast_convert_type(xv, jnp.int32)
            iv = 0x5F3759DF - (iv >> 1)
            y = lax.bitcast_convert_type(iv, jnp.float32)
            hx = xv * 0.5
            for _ in range(3):
                y = y * (1.5 - hx * y * y)
            for k in range(NK):
                erows[b][t, pl.ds(k * 16, 16)] = \
                    (e[k] - mean) * y * g[k] + bt[k]

    fetch(0, 0)

    def body2(ch, carry):
        for b in (0, 1):
            gi = 2 * ch + b
            nb = 1 - b

            @pl.when(gi + 1 < ROWS_W)
            def _():
                fetch(gi + 1, nb)

            pltpu.make_async_copy(ttab_hbm.at[idxt[b]], erows[b],
                                  sem_t[b]).wait()

            compute_chunk(b)
            pltpu.async_copy(erows[b], out_hbm.at[base_b + gi], sem_o[b])
        return carry

    lax.fori_loop(0, ROWS_W // 2, body2, 0)

    for b in (0, 1):
        pltpu.make_async_copy(erows[b], out_hbm.at[base_b], sem_o[b]).wait()


def kernel(input_token_id, input_position_id, token_table, pos_table,
           ln_gamma, ln_beta):
    tok = jnp.asarray(input_token_id, jnp.int32).reshape(N)
    pos = jnp.asarray(input_position_id, jnp.int32).reshape(N)
    ttab = jnp.pad(token_table, ((0, 0), (0, PAD - DIM)))
    ptab = jnp.pad(pos_table[:L], ((0, 0), (0, PAD - DIM)))
    out = _embed_ln_kernel(tok, pos, ttab, ptab, ln_gamma, ln_beta)
    return out[:, :, :DIM]
